# Initial kernel scaffold; baseline (speedup 1.0000x reference)
import jax
import jax.numpy as jnp
from jax.experimental import pallas as pl


def kernel(target, model_points, idx, H):
    raise NotImplementedError("write your pallas kernel here")


# speedup vs baseline: 2.6982x; 2.6982x over previous
import functools

import jax
import jax.numpy as jnp
from jax import lax
from jax.experimental import pallas as pl
from jax.experimental.pallas import tpu as pltpu
from jax.experimental.pallas import tpu_sc as plsc

_SYM = (0, 2, 5, 8)
_BS = 8
_NP = 2048
_L = 16
_NC = 2
_NS = 16
_NW = _NC * _NS
_QPW = _NP // _NW
_NRV = _NP // _L
_QG = 8
_F32_BIG = 3.0e38


def _bcast_lane(vec, k):
    idx = jnp.full((_L, 1), k, dtype=jnp.int32)
    return lax.gather(
        vec, idx,
        lax.GatherDimensionNumbers(
            offset_dims=(), collapsed_slice_dims=(0,), start_index_map=(0,)),
        (1,), mode=lax.GatherScatterMode.PROMISE_IN_BOUNDS)


def _sqrt16(x):
    xc = jnp.maximum(x, jnp.float32(1e-30))
    i = lax.bitcast_convert_type(xc, jnp.int32)
    y = lax.bitcast_convert_type(jnp.int32(0x5F3759DF) - (i >> 1), jnp.float32)
    half = jnp.float32(0.5) * xc
    for _ in range(3):
        y = y * (jnp.float32(1.5) - half * y * y)
    return jnp.maximum(x, jnp.float32(0.0)) * y


def _make_sc_kernel():
    mesh = plsc.VectorSubcoreMesh(core_axis_name="c", subcore_axis_name="s")

    @functools.partial(
        pl.kernel,
        mesh=mesh,
        compiler_params=pltpu.CompilerParams(needs_layout_passes=False),
        out_type=jax.ShapeDtypeStruct((_NW, _L), jnp.float32),
        scratch_types=[
            pltpu.VMEM((3, _NP), jnp.float32),
            pltpu.VMEM((_NP,), jnp.float32),
            pltpu.VMEM((3, _QPW), jnp.float32),
            pltpu.VMEM((4, _QPW), jnp.float32),
            pltpu.VMEM((_L,), jnp.float32),
            pltpu.VMEM((_L,), jnp.float32),
        ],
    )
    def sck(tt_hbm, mq_hbm, h_hbm, out_hbm, ref_v, rsq_v, mp_v, qd_v, h_v,
            res_v):
        cid = lax.axis_index("c")
        sid = lax.axis_index("s")
        wid = sid * _NC + cid
        qbase = wid * _QPW
        lanes = lax.iota(jnp.int32, _L)

        def batch_body(b, res):
            pltpu.sync_copy(tt_hbm.at[b], ref_v)
            pltpu.sync_copy(mq_hbm.at[b, wid], mp_v)
            pltpu.sync_copy(h_hbm.at[b], h_v)
            hv = h_v[:]
            r00 = _bcast_lane(hv, 0)
            r01 = _bcast_lane(hv, 1)
            r02 = _bcast_lane(hv, 2)
            tx = _bcast_lane(hv, 3)
            r10 = _bcast_lane(hv, 4)
            r11 = _bcast_lane(hv, 5)
            r12 = _bcast_lane(hv, 6)
            ty = _bcast_lane(hv, 7)
            r20 = _bcast_lane(hv, 8)
            r21 = _bcast_lane(hv, 9)
            r22 = _bcast_lane(hv, 10)
            tz = _bcast_lane(hv, 11)

            def rsq_body(j, carry):
                o = j * _L
                rx = ref_v[0, pl.ds(o, _L)]
                ry = ref_v[1, pl.ds(o, _L)]
                rz = ref_v[2, pl.ds(o, _L)]
                rsq_v[pl.ds(o, _L)] = rx * rx + ry * ry + rz * rz
                return carry

            lax.fori_loop(0, _NRV, rsq_body, 0)

            diag = jnp.zeros((_L,), jnp.float32)
            for k in range(_QPW // _L):
                o = k * _L
                mx = mp_v[0, pl.ds(o, _L)]
                my = mp_v[1, pl.ds(o, _L)]
                mz = mp_v[2, pl.ds(o, _L)]
                tfx = r00 * mx + r01 * my + r02 * mz + tx
                tfy = r10 * mx + r11 * my + r12 * mz + ty
                tfz = r20 * mx + r21 * my + r22 * mz + tz
                qd_v[0, pl.ds(o, _L)] = jnp.float32(-2.0) * tfx
                qd_v[1, pl.ds(o, _L)] = jnp.float32(-2.0) * tfy
                qd_v[2, pl.ds(o, _L)] = jnp.float32(-2.0) * tfz
                qd_v[3, pl.ds(o, _L)] = tfx * tfx + tfy * tfy + tfz * tfz
                gx = ref_v[0, pl.ds(qbase + o, _L)]
                gy = ref_v[1, pl.ds(qbase + o, _L)]
                gz = ref_v[2, pl.ds(qbase + o, _L)]
                dx = tfx - gx
                dy = tfy - gy
                dz = tfz - gz
                diag = diag + _sqrt16(dx * dx + dy * dy + dz * dz)

            msum = jnp.zeros((_L,), jnp.float32)
            for k in range(_QPW // _L):
                o = k * _L
                n2x = qd_v[0, pl.ds(o, _L)]
                n2y = qd_v[1, pl.ds(o, _L)]
                n2z = qd_v[2, pl.ds(o, _L)]
                minvec = jnp.full((_L,), _F32_BIG, jnp.float32)
                for h in range(_L // _QG):
                    bxs = [_bcast_lane(n2x, h * _QG + q) for q in range(_QG)]
                    bys = [_bcast_lane(n2y, h * _QG + q) for q in range(_QG)]
                    bzs = [_bcast_lane(n2z, h * _QG + q) for q in range(_QG)]

                    def nn_body(j, accs, bxs=bxs, bys=bys, bzs=bzs):
                        o2 = j * (2 * _L)
                        new = list(accs)
                        for u in range(2):
                            oo = o2 + u * _L
                            rx = ref_v[0, pl.ds(oo, _L)]
                            ry = ref_v[1, pl.ds(oo, _L)]
                            rz = ref_v[2, pl.ds(oo, _L)]
                            rq = rsq_v[pl.ds(oo, _L)]
                            for q in range(_QG):
                                d2 = rx * bxs[q] + ry * bys[q] + rz * bzs[q] + rq
                                new[q] = jnp.minimum(new[q], d2)
                        return tuple(new)

                    accs = lax.fori_loop(
                        0, _NRV // 2, nn_body,
                        tuple(jnp.full((_L,), _F32_BIG, jnp.float32)
                              for _ in range(_QG)))
                    for q in range(_QG):
                        m = jnp.min(accs[q])
                        minvec = jnp.where(lanes == (h * _QG + q), m, minvec)
                qsq = qd_v[3, pl.ds(o, _L)]
                msum = msum + _sqrt16(minvec + qsq)

            res = jnp.where(lanes == b, jnp.sum(msum), res)
            res = jnp.where(lanes == (b + _BS), jnp.sum(diag), res)
            return res

        res = lax.fori_loop(0, _BS, batch_body, jnp.zeros((_L,), jnp.float32))
        res_v[:] = res
        pltpu.sync_copy(res_v, out_hbm.at[wid])

    return sck


_SC_KERNEL = _make_sc_kernel()


def kernel(target, model_points, idx, H):
    tt = jnp.transpose(target, (0, 2, 1))
    mq = jnp.transpose(model_points, (0, 2, 1))
    mq = jnp.transpose(mq.reshape(_BS, 3, _NW, _QPW), (0, 2, 1, 3))
    hf = H.reshape(_BS, _L)
    parts = _SC_KERNEL(tt, mq, hf)
    sums = jnp.sum(parts, axis=0) / jnp.float32(_NP)
    dmin = sums[:_BS]
    ddiag = sums[_BS:]
    sym = jnp.asarray(_SYM, dtype=idx.dtype)
    is_sym = jnp.any(idx[:, 0, None] == sym[None, :], axis=1)
    return jnp.where(is_sym, dmin, ddiag)
